# baseline (speedup 1.0000x reference)
import jax
import jax.numpy as jnp
from jax import lax
from jax.experimental import pallas as pl
from jax.experimental.pallas import tpu as pltpu
from jax.experimental.pallas import tpu_sc as plsc

N_SENT = 4096
S = 50
SP = 56
D = 128
NC, NS = 2, 16
NW = NC * NS
SENT_PER_W = N_SENT // NW
SENT_PER_CHUNK = 2
C = SENT_PER_CHUNK * SP
NCHUNK = SENT_PER_W // SENT_PER_CHUNK


def _emb_body(idx_hbm, table_hbm, out_hbm, idx_v, buf0, buf1, g0, g1, o0, o1):
    wid = lax.axis_index("s") * NC + lax.axis_index("c")
    pltpu.sync_copy(idx_hbm.at[wid], idx_v)
    sent_base = wid * SENT_PER_W

    bufs = (buf0, buf1)
    gsems = (g0, g1)
    osems = (o0, o1)

    def gather(c, b):
        pltpu.async_copy(table_hbm.at[idx_v.at[c]], bufs[b], gsems[b])

    def wait_gather(c, b):
        pltpu.make_async_copy(table_hbm.at[idx_v.at[c]], bufs[b], gsems[b]).wait()

    def store(c, b):
        sent0 = sent_base + c * SENT_PER_CHUNK
        pltpu.async_copy(bufs[b], out_hbm.at[pl.ds(sent0 * SP, C)], osems[b])

    def wait_store(b):
        pltpu.make_async_copy(bufs[b], out_hbm.at[pl.ds(0, C)], osems[b]).wait()

    gather(0, 0)

    def body(g, carry):
        c0 = g * 2
        wait_gather(c0, 0)
        store(c0, 0)

        @pl.when(g >= 1)
        def _():
            wait_store(1)

        gather(c0 + 1, 1)

        wait_gather(c0 + 1, 1)
        store(c0 + 1, 1)
        wait_store(0)

        @pl.when(g < NCHUNK // 2 - 1)
        def _():
            gather(c0 + 2, 0)

        return carry

    lax.fori_loop(0, NCHUNK // 2, body, 0)
    wait_store(1)


def _run(idx_grp, table):
    f = pl.kernel(
        _emb_body,
        out_type=jax.ShapeDtypeStruct((N_SENT * SP, D), jnp.float32),
        mesh=plsc.VectorSubcoreMesh(core_axis_name="c", subcore_axis_name="s"),
        scratch_types=[
            pltpu.VMEM((NCHUNK, C), jnp.int32),
            pltpu.VMEM((C, D), jnp.float32),
            pltpu.VMEM((C, D), jnp.float32),
            pltpu.SemaphoreType.DMA,
            pltpu.SemaphoreType.DMA,
            pltpu.SemaphoreType.DMA,
            pltpu.SemaphoreType.DMA,
        ],
    )
    return f(idx_grp, table)


def kernel(idx, emb_weight):
    idx_pad = jnp.pad(idx.astype(jnp.int32), ((0, 0), (0, SP - S)))
    idx_grp = idx_pad.reshape(NW, NCHUNK, C)
    out = _run(idx_grp, emb_weight)
    return out.reshape(N_SENT, SP, D)[:, :S, :]


# speedup vs baseline: 6.7624x; 6.7624x over previous
import jax
import jax.numpy as jnp
from jax import lax
from jax.experimental import pallas as pl
from jax.experimental.pallas import tpu as pltpu
from jax.experimental.pallas import tpu_sc as plsc

N_SENT = 4096
S = 50
SP = 56
D = 128
NC, NS = 2, 16
NW = NC * NS
SENT_PER_W = N_SENT // NW
SENT_PER_CHUNK = 2
C = SENT_PER_CHUNK * S
NCHUNK = SENT_PER_W // SENT_PER_CHUNK


def _emb_body(idx_hbm, table_hbm, out_hbm, idx_v, buf0, buf1, g0, g1, o0, o1):
    wid = lax.axis_index("s") * NC + lax.axis_index("c")
    pltpu.sync_copy(idx_hbm.at[wid], idx_v)
    sent_base = wid * SENT_PER_W

    bufs = (buf0, buf1)
    gsems = (g0, g1)
    osems = (o0, o1)

    def gather(c, b):
        pltpu.async_copy(table_hbm.at[idx_v.at[c]], bufs[b], gsems[b])

    def wait_gather(c, b):
        pltpu.make_async_copy(table_hbm.at[idx_v.at[c]], bufs[b], gsems[b]).wait()

    def store(c, b):
        sent0 = sent_base + c * SENT_PER_CHUNK
        for i in range(SENT_PER_CHUNK):
            pltpu.async_copy(
                bufs[b].at[pl.ds(i * S, S)], out_hbm.at[sent0 + i], osems[b]
            )

    def wait_store(b):
        for _ in range(SENT_PER_CHUNK):
            pltpu.make_async_copy(
                bufs[b].at[pl.ds(0, S)], out_hbm.at[0], osems[b]
            ).wait()

    gather(0, 0)

    def body(g, carry):
        c0 = g * 2
        wait_gather(c0, 0)
        store(c0, 0)

        @pl.when(g >= 1)
        def _():
            wait_store(1)

        gather(c0 + 1, 1)

        wait_gather(c0 + 1, 1)
        store(c0 + 1, 1)
        wait_store(0)

        @pl.when(g < NCHUNK // 2 - 1)
        def _():
            gather(c0 + 2, 0)

        return carry

    lax.fori_loop(0, NCHUNK // 2, body, 0)
    wait_store(1)


def _run(idx_grp, table):
    f = pl.kernel(
        _emb_body,
        out_type=jax.ShapeDtypeStruct((N_SENT, S, D), jnp.float32),
        mesh=plsc.VectorSubcoreMesh(core_axis_name="c", subcore_axis_name="s"),
        scratch_types=[
            pltpu.VMEM((NCHUNK, C), jnp.int32),
            pltpu.VMEM((C, D), jnp.float32),
            pltpu.VMEM((C, D), jnp.float32),
            pltpu.SemaphoreType.DMA,
            pltpu.SemaphoreType.DMA,
            pltpu.SemaphoreType.DMA,
            pltpu.SemaphoreType.DMA,
        ],
    )
    return f(idx_grp, table)


def kernel(idx, emb_weight):
    idx_grp = idx.astype(jnp.int32).reshape(NW, NCHUNK, C)
    return _run(idx_grp, emb_weight)
